# baseline (speedup 1.0000x reference)
import functools

import jax
import jax.numpy as jnp
from jax.experimental import pallas as pl
from jax.experimental.pallas import tpu as pltpu

B = 4
N = 5000
NP = 5120
T = 128
NT = NP // T
C = 92
CP = 128
NMS_THRESH = 0.7
NMS_REMOVE = 0.01
K_OUT = 100

_HI = jax.lax.Precision.HIGHEST


def _dot(a, b, dims):
    return jax.lax.dot_general(a, b, (dims, ((), ())), precision=_HI,
                               preferred_element_type=jnp.float32)


def _postprocess_kernel(logits_ref, boxes_ref, scores_ref, labels_ref,
                        boxesT_ref, data_cols, sorted_cols, sorted_rows,
                        misc, colstore, nms_rows):
    f32 = jnp.float32
    ident = (jax.lax.broadcasted_iota(jnp.int32, (T, T), 0) ==
             jax.lax.broadcasted_iota(jnp.int32, (T, T), 1)).astype(f32)

    def col2row(v):
        return _dot(v, ident, ((0,), (0,)))

    def row2col(r):
        return _dot(ident, r, ((1,), (1,)))

    iota_col_T = jax.lax.broadcasted_iota(jnp.int32, (T, 1), 0).astype(f32)
    lane_row_NP = jax.lax.broadcasted_iota(jnp.int32, (1, NP), 1).astype(f32)

    l = logits_ref[0]
    lane = jax.lax.broadcasted_iota(jnp.int32, (NP, CP), 1)
    mask_all = lane < C
    mask_fg = lane < C - 1
    neg = jnp.float32(-jnp.inf)
    m_all = jnp.max(jnp.where(mask_all, l, neg), axis=1, keepdims=True)
    e = jnp.where(mask_all, jnp.exp(l - m_all), 0.0)
    s_sum = jnp.sum(e, axis=1, keepdims=True)
    m_fg = jnp.max(jnp.where(mask_fg, l, neg), axis=1, keepdims=True)
    score_col = jnp.exp(m_fg - m_all) / s_sum
    is_max = (l == m_fg) & mask_fg
    label_col = jnp.min(jnp.where(is_max, lane, 10000), axis=1,
                        keepdims=True).astype(f32)
    q_valid = jax.lax.broadcasted_iota(jnp.int32, (NP, 1), 0) < N
    score_col = jnp.where(q_valid, score_col, -1.0)
    label_col = jnp.where(q_valid, label_col, 999.0)
    colstore[:, 0:1] = score_col
    colstore[:, 1:2] = label_col

    b = boxes_ref[0]
    cx, cy, w, h = b[0:1, :], b[1:2, :], b[2:3, :], b[3:4, :]
    data_cols[0:1, :] = cx - 0.5 * w
    data_cols[1:2, :] = cy - 0.5 * h
    data_cols[2:3, :] = cx + 0.5 * w
    data_cols[3:4, :] = cy + 0.5 * h
    data_cols[6:7, :] = lane_row_NP
    data_cols[7:8, :] = jnp.zeros((1, NP), f32)

    def xpose_body(t, _):
        base = t * T
        cc = colstore[pl.ds(base, T), 0:2]
        data_cols[4:6, pl.ds(base, T)] = col2row(cc)
        return 0

    jax.lax.fori_loop(0, NT, xpose_body, 0)

    s_row = data_cols[4:5, :]

    def rank_body(t, _):
        base = t * T
        s_chunk = row2col(data_cols[4:5, pl.ds(base, T)])
        i_orig = base + iota_col_T
        before = (s_row > s_chunk) | ((s_row == s_chunk) &
                                      (lane_row_NP < i_orig))
        rank_col = jnp.sum(before.astype(f32), axis=1, keepdims=True)
        misc[0:1, pl.ds(base, T)] = col2row(rank_col)
        return 0

    jax.lax.fori_loop(0, NT, rank_body, 0)

    rank_row = misc[0:1, :]

    def sort_body(t, _):
        base = t * T
        r_col = base + iota_col_T
        onehot = (rank_row == r_col).astype(f32)
        chunk = _dot(onehot, data_cols[:, :], ((1,), (1,)))
        sorted_rows[pl.ds(base, T), :] = chunk
        sorted_cols[:, pl.ds(base, T)] = col2row(chunk)
        return 0

    jax.lax.fori_loop(0, NT, sort_body, 0)

    mc = jnp.max(data_cols[0:4, :])
    sc = sorted_cols[:, :]
    offs_r = sc[5:6, :] * (mc + 1.0)
    x1o, y1o = sc[0:1, :] + offs_r, sc[1:2, :] + offs_r
    x2o, y2o = sc[2:3, :] + offs_r, sc[3:4, :] + offs_r
    nms_rows[0:1, :] = x1o
    nms_rows[1:2, :] = y1o
    nms_rows[2:3, :] = x2o
    nms_rows[3:4, :] = y2o
    nms_rows[4:5, :] = (x2o - x1o) * (y2o - y1o)
    misc[1:2, :] = (lane_row_NP >= N).astype(f32)

    CW = 1024
    lane_row_CW = jax.lax.broadcasted_iota(jnp.int32, (1, CW), 1)

    def nms_body(t, _):
        base = t * T
        trows = sorted_rows[pl.ds(base, T), :]
        offs_c = trows[:, 5:6] * (mc + 1.0)
        x1c, y1c = trows[:, 0:1] + offs_c, trows[:, 1:2] + offs_c
        x2c, y2c = trows[:, 2:3] + offs_c, trows[:, 3:4] + offs_c
        area_c = (x2c - x1c) * (y2c - y1c)

        def chunk_body(c, cacc):
            cb = c * CW
            x1j = nms_rows[0:1, pl.ds(cb, CW)]
            y1j = nms_rows[1:2, pl.ds(cb, CW)]
            x2j = nms_rows[2:3, pl.ds(cb, CW)]
            y2j = nms_rows[3:4, pl.ds(cb, CW)]
            area_j = nms_rows[4:5, pl.ds(cb, CW)]
            supp_j = misc[1:2, pl.ds(cb, CW)]
            iw = jnp.maximum(jnp.minimum(x2c, x2j) - jnp.maximum(x1c, x1j),
                             0.0)
            ih = jnp.maximum(jnp.minimum(y2c, y2j) - jnp.maximum(y1c, y1j),
                             0.0)
            inter = iw * ih
            union = jnp.maximum(area_c + area_j - inter, 1e-9)
            m = ((inter > NMS_THRESH * union) & (cb + lane_row_CW < base) &
                 (supp_j < 0.5))
            return jnp.maximum(cacc, jnp.max(m.astype(f32), axis=1,
                                             keepdims=True))

        nchunks = (base + CW - 1) // CW
        cross_col = jax.lax.fori_loop(0, nchunks, chunk_body,
                                      jnp.zeros((T, 1), f32))

        tcols = nms_rows[0:4, pl.ds(base, T)]
        x1t, y1t, x2t, y2t = (tcols[0:1, :], tcols[1:2, :],
                              tcols[2:3, :], tcols[3:4, :])
        area_t = nms_rows[4:5, pl.ds(base, T)]
        iwt = jnp.maximum(jnp.minimum(x2c, x2t) - jnp.maximum(x1c, x1t), 0.0)
        iht = jnp.maximum(jnp.minimum(y2c, y2t) - jnp.maximum(y1c, y1t), 0.0)
        intert = iwt * iht
        uniont = jnp.maximum(area_c + area_t - intert, 1e-9)
        lt_mask = (jax.lax.broadcasted_iota(jnp.int32, (T, T), 1) <
                   jax.lax.broadcasted_iota(jnp.int32, (T, T), 0))
        a_t = ((intert > NMS_THRESH * uniont) & lt_mask).astype(f32)

        s0 = jnp.maximum(cross_col,
                         row2col(misc[1:2, pl.ds(base, T)]))

        def fx_cond(carry):
            return carry[1]

        def fx_body(carry):
            s, _ = carry
            cnt = _dot(a_t, 1.0 - s, ((1,), (0,)))
            s_new = jnp.maximum(s0, (cnt > 0.5).astype(f32))
            return s_new, jnp.any(s_new != s)

        s_fin, _ = jax.lax.while_loop(fx_cond, fx_body, (s0, True))
        misc[1:2, pl.ds(base, T)] = col2row(s_fin)
        return 0

    jax.lax.fori_loop(0, NT, nms_body, 0)

    alive_row = 1.0 - misc[1:2, :]
    upper = (jax.lax.broadcasted_iota(jnp.int32, (T, T), 0) <=
             jax.lax.broadcasted_iota(jnp.int32, (T, T), 1)).astype(f32)

    def csum_body(t, off):
        base = t * T
        chunk = 1.0 - misc[1:2, pl.ds(base, T)]
        pre = _dot(chunk, upper, ((1,), (0,))) + off
        misc[4:5, pl.ds(base, T)] = pre
        return pre[:, T - 1:T]

    k_tot = jax.lax.fori_loop(0, NT, csum_body, jnp.zeros((1, 1), f32))

    misc[3:4, :] = jnp.zeros((1, NP), f32)

    def ko_body(t, _):
        base = t * T
        r_col = base + iota_col_T
        onehot = (misc[0:1, :] == r_col).astype(f32)
        al_row = 1.0 - misc[1:2, pl.ds(base, T)]
        misc[3:4, :] += _dot(al_row, onehot, ((1,), (0,)))
        return 0

    jax.lax.fori_loop(0, NT, ko_body, 0)
    ko_row = misc[3:4, :]
    nk_row = 1.0 - ko_row

    def dsum_body(t, off):
        base = t * T
        chunk = 1.0 - misc[3:4, pl.ds(base, T)]
        pre = _dot(chunk, upper, ((1,), (0,))) + off
        misc[5:6, pl.ds(base, T)] = pre
        return pre[:, T - 1:T]

    jax.lax.fori_loop(0, NT, dsum_body, jnp.zeros((1, 1), f32))

    c_row = misc[4:5, :]
    d_row = misc[5:6, :]
    k_col = iota_col_T
    match1 = ((alive_row > 0.5) & (c_row == k_col + 1.0)).astype(f32)
    part1 = _dot(match1, sorted_cols[:, :], ((1,), (1,)))
    match2 = ((ko_row < 0.5) & (d_row == k_col - k_tot + 1.0)).astype(f32)
    part2 = _dot(match2, data_cols[:, :], ((1,), (1,)))

    boxes_out = part1[:, 0:4] + part2[:, 0:4]
    score_out = part1[:, 4:5] + NMS_REMOVE * part2[:, 4:5]
    label_out = part1[:, 5:6] + part2[:, 5:6]

    scores_ref[0] = col2row(score_out)
    labels_ref[0] = jnp.round(col2row(label_out)).astype(jnp.int32)
    boxesT_ref[0, 0:4, :] = col2row(boxes_out)
    boxesT_ref[0, 4:8, :] = jnp.zeros((4, T), f32)


@jax.jit
def kernel(pred_logits, pred_boxes, target_sizes):
    f32 = jnp.float32
    logits = jnp.pad(pred_logits, ((0, 0), (0, NP - N), (0, CP - C)))
    boxesT = jnp.pad(jnp.transpose(pred_boxes, (0, 2, 1)),
                     ((0, 0), (0, 4), (0, NP - N)))

    scores_o, labels_o, boxesT_o = pl.pallas_call(
        _postprocess_kernel,
        grid=(B,),
        in_specs=[
            pl.BlockSpec((1, NP, CP), lambda i: (i, 0, 0)),
            pl.BlockSpec((1, 8, NP), lambda i: (i, 0, 0)),
        ],
        out_specs=[
            pl.BlockSpec((1, 1, T), lambda i: (i, 0, 0)),
            pl.BlockSpec((1, 1, T), lambda i: (i, 0, 0)),
            pl.BlockSpec((1, 8, T), lambda i: (i, 0, 0)),
        ],
        out_shape=[
            jax.ShapeDtypeStruct((B, 1, T), f32),
            jax.ShapeDtypeStruct((B, 1, T), jnp.int32),
            jax.ShapeDtypeStruct((B, 8, T), f32),
        ],
        scratch_shapes=[
            pltpu.VMEM((8, NP), f32),
            pltpu.VMEM((8, NP), f32),
            pltpu.VMEM((NP, 8), f32),
            pltpu.VMEM((8, NP), f32),
            pltpu.VMEM((NP, 8), f32),
            pltpu.VMEM((8, NP), f32),
        ],
        compiler_params=pltpu.CompilerParams(
            dimension_semantics=("arbitrary",)),
    )(logits, boxesT)

    scores = scores_o[:, 0, :K_OUT]
    labels = labels_o[:, 0, :K_OUT]
    boxes = jnp.transpose(boxesT_o[:, 0:4, :K_OUT], (0, 2, 1))
    img_h = target_sizes[:, 0].astype(f32)
    img_w = target_sizes[:, 1].astype(f32)
    scale = jnp.stack([img_w, img_h, img_w, img_h], axis=1)
    boxes = boxes * scale[:, None, :]
    return scores, labels, boxes


# speedup vs baseline: 1.1157x; 1.1157x over previous
import functools

import jax
import jax.numpy as jnp
from jax.experimental import pallas as pl
from jax.experimental.pallas import tpu as pltpu

B = 4
N = 5000
NP = 5120
T = 128
NT = NP // T
C = 92
CP = 128
NMS_THRESH = 0.7
NMS_REMOVE = 0.01
K_OUT = 100

_HI = jax.lax.Precision.HIGHEST


def _dot(a, b, dims):
    return jax.lax.dot_general(a, b, (dims, ((), ())), precision=_HI,
                               preferred_element_type=jnp.float32)


def _postprocess_kernel(logits_ref, boxes_ref, scores_ref, labels_ref,
                        boxesT_ref, data_cols, sorted_cols, sorted_rows,
                        misc, colstore, nms_rows):
    f32 = jnp.float32
    ident = (jax.lax.broadcasted_iota(jnp.int32, (T, T), 0) ==
             jax.lax.broadcasted_iota(jnp.int32, (T, T), 1)).astype(f32)

    def col2row(v):
        return _dot(v, ident, ((0,), (0,)))

    def row2col(r):
        return _dot(ident, r, ((1,), (1,)))

    iota_col_T = jax.lax.broadcasted_iota(jnp.int32, (T, 1), 0).astype(f32)
    lane_row_NP = jax.lax.broadcasted_iota(jnp.int32, (1, NP), 1).astype(f32)

    l = logits_ref[0]
    lane = jax.lax.broadcasted_iota(jnp.int32, (NP, CP), 1)
    mask_all = lane < C
    mask_fg = lane < C - 1
    neg = jnp.float32(-jnp.inf)
    m_all = jnp.max(jnp.where(mask_all, l, neg), axis=1, keepdims=True)
    e = jnp.where(mask_all, jnp.exp(l - m_all), 0.0)
    s_sum = jnp.sum(e, axis=1, keepdims=True)
    m_fg = jnp.max(jnp.where(mask_fg, l, neg), axis=1, keepdims=True)
    score_col = jnp.exp(m_fg - m_all) / s_sum
    is_max = (l == m_fg) & mask_fg
    label_col = jnp.min(jnp.where(is_max, lane, 10000), axis=1,
                        keepdims=True).astype(f32)
    q_valid = jax.lax.broadcasted_iota(jnp.int32, (NP, 1), 0) < N
    score_col = jnp.where(q_valid, score_col, -1.0)
    label_col = jnp.where(q_valid, label_col, 999.0)
    colstore[:, 0:1] = score_col
    colstore[:, 1:2] = label_col

    b = boxes_ref[0]
    cx, cy, w, h = b[0:1, :], b[1:2, :], b[2:3, :], b[3:4, :]
    data_cols[0:1, :] = cx - 0.5 * w
    data_cols[1:2, :] = cy - 0.5 * h
    data_cols[2:3, :] = cx + 0.5 * w
    data_cols[3:4, :] = cy + 0.5 * h
    data_cols[6:7, :] = lane_row_NP
    data_cols[7:8, :] = jnp.zeros((1, NP), f32)

    def xpose_body(t, _):
        base = t * T
        cc = colstore[pl.ds(base, T), 0:2]
        data_cols[4:6, pl.ds(base, T)] = col2row(cc)
        return 0

    jax.lax.fori_loop(0, NT, xpose_body, 0)

    s_row = data_cols[4:5, :]

    def rank_body(t, _):
        base = t * T
        s_chunk = row2col(data_cols[4:5, pl.ds(base, T)])
        i_orig = base + iota_col_T
        gt = (s_row > s_chunk).astype(f32)
        tie = ((s_row == s_chunk) & (lane_row_NP < i_orig)).astype(f32)
        rank_col = jnp.sum(gt, axis=1, keepdims=True) + \
            jnp.sum(tie, axis=1, keepdims=True)
        misc[0:1, pl.ds(base, T)] = col2row(rank_col)
        return 0

    jax.lax.fori_loop(0, NT, rank_body, 0)

    rank_row = misc[0:1, :]

    def sort_body(t, _):
        base = t * T
        r_col = base + iota_col_T
        onehot = (rank_row == r_col).astype(f32)
        chunk = _dot(onehot, data_cols[:, :], ((1,), (1,)))
        sorted_rows[pl.ds(base, T), :] = chunk
        sorted_cols[:, pl.ds(base, T)] = col2row(chunk)
        return 0

    jax.lax.fori_loop(0, NT, sort_body, 0)

    mc = jnp.max(data_cols[0:4, :])
    sc = sorted_cols[:, :]
    offs_r = sc[5:6, :] * (mc + 1.0)
    x1o, y1o = sc[0:1, :] + offs_r, sc[1:2, :] + offs_r
    x2o, y2o = sc[2:3, :] + offs_r, sc[3:4, :] + offs_r
    nms_rows[0:1, :] = x1o
    nms_rows[1:2, :] = y1o
    nms_rows[2:3, :] = x2o
    nms_rows[3:4, :] = y2o
    nms_rows[4:5, :] = (x2o - x1o) * (y2o - y1o)
    misc[1:2, :] = (lane_row_NP >= N).astype(f32)

    CW = 1024
    lane_row_CW = jax.lax.broadcasted_iota(jnp.int32, (1, CW), 1)

    def nms_body(t, _):
        base = t * T
        trows = sorted_rows[pl.ds(base, T), :]
        offs_c = trows[:, 5:6] * (mc + 1.0)
        x1c, y1c = trows[:, 0:1] + offs_c, trows[:, 1:2] + offs_c
        x2c, y2c = trows[:, 2:3] + offs_c, trows[:, 3:4] + offs_c
        area_c = (x2c - x1c) * (y2c - y1c)

        def chunk_body(c, cacc):
            cb = c * CW
            x1j = nms_rows[0:1, pl.ds(cb, CW)]
            y1j = nms_rows[1:2, pl.ds(cb, CW)]
            x2j = nms_rows[2:3, pl.ds(cb, CW)]
            y2j = nms_rows[3:4, pl.ds(cb, CW)]
            area_j = nms_rows[4:5, pl.ds(cb, CW)]
            supp_j = misc[1:2, pl.ds(cb, CW)]
            iw = jnp.maximum(jnp.minimum(x2c, x2j) - jnp.maximum(x1c, x1j),
                             0.0)
            ih = jnp.maximum(jnp.minimum(y2c, y2j) - jnp.maximum(y1c, y1j),
                             0.0)
            inter = iw * ih
            union = jnp.maximum(area_c + area_j - inter, 1e-9)
            m = ((inter / union > NMS_THRESH) & (cb + lane_row_CW < base) &
                 (supp_j < 0.5))
            return jnp.maximum(cacc, jnp.max(m.astype(f32), axis=1,
                                             keepdims=True))

        nchunks = (base + CW - 1) // CW
        cross_col = jax.lax.fori_loop(0, nchunks, chunk_body,
                                      jnp.zeros((T, 1), f32))

        tcols = nms_rows[0:4, pl.ds(base, T)]
        x1t, y1t, x2t, y2t = (tcols[0:1, :], tcols[1:2, :],
                              tcols[2:3, :], tcols[3:4, :])
        area_t = nms_rows[4:5, pl.ds(base, T)]
        iwt = jnp.maximum(jnp.minimum(x2c, x2t) - jnp.maximum(x1c, x1t), 0.0)
        iht = jnp.maximum(jnp.minimum(y2c, y2t) - jnp.maximum(y1c, y1t), 0.0)
        intert = iwt * iht
        uniont = jnp.maximum(area_c + area_t - intert, 1e-9)
        lt_mask = (jax.lax.broadcasted_iota(jnp.int32, (T, T), 1) <
                   jax.lax.broadcasted_iota(jnp.int32, (T, T), 0))
        a_t = ((intert / uniont > NMS_THRESH) & lt_mask).astype(f32)

        s0 = jnp.maximum(cross_col,
                         row2col(misc[1:2, pl.ds(base, T)]))

        def fx_cond(carry):
            return carry[1]

        def fx_body(carry):
            s, _ = carry
            cnt = _dot(a_t, 1.0 - s, ((1,), (0,)))
            s_new = jnp.maximum(s0, (cnt > 0.5).astype(f32))
            return s_new, jnp.any(s_new != s)

        s_fin, _ = jax.lax.while_loop(fx_cond, fx_body, (s0, True))
        misc[1:2, pl.ds(base, T)] = col2row(s_fin)
        return 0

    jax.lax.fori_loop(0, NT, nms_body, 0)

    alive_row = 1.0 - misc[1:2, :]
    upper = (jax.lax.broadcasted_iota(jnp.int32, (T, T), 0) <=
             jax.lax.broadcasted_iota(jnp.int32, (T, T), 1)).astype(f32)

    def csum_body(t, off):
        base = t * T
        chunk = 1.0 - misc[1:2, pl.ds(base, T)]
        pre = _dot(chunk, upper, ((1,), (0,))) + off
        misc[4:5, pl.ds(base, T)] = pre
        return pre[:, T - 1:T]

    k_tot = jax.lax.fori_loop(0, NT, csum_body, jnp.zeros((1, 1), f32))

    misc[3:4, :] = jnp.zeros((1, NP), f32)

    def ko_body(t, _):
        base = t * T
        oi_col = sorted_rows[pl.ds(base, T), 6:7]
        al_col = row2col(1.0 - misc[1:2, pl.ds(base, T)])
        contrib = jnp.max((oi_col == lane_row_NP).astype(f32) * al_col,
                          axis=0, keepdims=True)
        misc[3:4, :] = jnp.maximum(misc[3:4, :], contrib)
        return 0

    jax.lax.fori_loop(0, NT, ko_body, 0)
    ko_row = misc[3:4, :]
    nk_row = 1.0 - ko_row

    def dsum_body(t, off):
        base = t * T
        chunk = 1.0 - misc[3:4, pl.ds(base, T)]
        pre = _dot(chunk, upper, ((1,), (0,))) + off
        misc[5:6, pl.ds(base, T)] = pre
        return pre[:, T - 1:T]

    jax.lax.fori_loop(0, NT, dsum_body, jnp.zeros((1, 1), f32))

    c_row = misc[4:5, :]
    d_row = misc[5:6, :]
    k_col = iota_col_T
    match1 = ((alive_row > 0.5) & (c_row == k_col + 1.0)).astype(f32)
    part1 = _dot(match1, sorted_cols[:, :], ((1,), (1,)))
    match2 = ((ko_row < 0.5) & (d_row == k_col - k_tot + 1.0)).astype(f32)
    part2 = _dot(match2, data_cols[:, :], ((1,), (1,)))

    boxes_out = part1[:, 0:4] + part2[:, 0:4]
    score_out = part1[:, 4:5] + NMS_REMOVE * part2[:, 4:5]
    label_out = part1[:, 5:6] + part2[:, 5:6]

    scores_ref[0] = col2row(score_out)
    labels_ref[0] = jnp.round(col2row(label_out)).astype(jnp.int32)
    boxesT_ref[0, 0:4, :] = col2row(boxes_out)
    boxesT_ref[0, 4:8, :] = jnp.zeros((4, T), f32)


@jax.jit
def kernel(pred_logits, pred_boxes, target_sizes):
    f32 = jnp.float32
    logits = jnp.pad(pred_logits, ((0, 0), (0, NP - N), (0, CP - C)))
    boxesT = jnp.pad(jnp.transpose(pred_boxes, (0, 2, 1)),
                     ((0, 0), (0, 4), (0, NP - N)))

    scores_o, labels_o, boxesT_o = pl.pallas_call(
        _postprocess_kernel,
        grid=(B,),
        in_specs=[
            pl.BlockSpec((1, NP, CP), lambda i: (i, 0, 0)),
            pl.BlockSpec((1, 8, NP), lambda i: (i, 0, 0)),
        ],
        out_specs=[
            pl.BlockSpec((1, 1, T), lambda i: (i, 0, 0)),
            pl.BlockSpec((1, 1, T), lambda i: (i, 0, 0)),
            pl.BlockSpec((1, 8, T), lambda i: (i, 0, 0)),
        ],
        out_shape=[
            jax.ShapeDtypeStruct((B, 1, T), f32),
            jax.ShapeDtypeStruct((B, 1, T), jnp.int32),
            jax.ShapeDtypeStruct((B, 8, T), f32),
        ],
        scratch_shapes=[
            pltpu.VMEM((8, NP), f32),
            pltpu.VMEM((8, NP), f32),
            pltpu.VMEM((NP, 8), f32),
            pltpu.VMEM((8, NP), f32),
            pltpu.VMEM((NP, 8), f32),
            pltpu.VMEM((8, NP), f32),
        ],
        compiler_params=pltpu.CompilerParams(
            dimension_semantics=("arbitrary",)),
    )(logits, boxesT)

    scores = scores_o[:, 0, :K_OUT]
    labels = labels_o[:, 0, :K_OUT]
    boxes = jnp.transpose(boxesT_o[:, 0:4, :K_OUT], (0, 2, 1))
    img_h = target_sizes[:, 0].astype(f32)
    img_w = target_sizes[:, 1].astype(f32)
    scale = jnp.stack([img_w, img_h, img_w, img_h], axis=1)
    boxes = boxes * scale[:, None, :]
    return scores, labels, boxes
